# Initial kernel scaffold; baseline (speedup 1.0000x reference)
import jax
import jax.numpy as jnp
from jax.experimental import pallas as pl


def kernel(positions, positional_embeddings):
    raise NotImplementedError("write your pallas kernel here")


# speedup vs baseline: 2.1901x; 2.1901x over previous
import functools

import jax
import jax.numpy as jnp
from jax import lax
from jax.experimental import pallas as pl
from jax.experimental.pallas import tpu as pltpu
from jax.experimental.pallas import tpu_sc as plsc

N_CORES = 2
N_SUBCORES = 16
N_WORKERS = N_CORES * N_SUBCORES

D = 1024
B = 4 * 8192
B_PER_W = B // N_WORKERS
CHUNK = 64
N_CHUNKS = B_PER_W // CHUNK


def _sc_gather(table, idx):
    mesh = plsc.VectorSubcoreMesh(core_axis_name="c", subcore_axis_name="s")

    @functools.partial(
        pl.kernel,
        mesh=mesh,
        out_type=jax.ShapeDtypeStruct((B, D), jnp.float32),
        scratch_types=[
            pltpu.VMEM((B_PER_W,), jnp.int32),
            pltpu.VMEM((CHUNK, D), jnp.float32),
            pltpu.SemaphoreType.DMA,
        ],
    )
    def k(table_hbm, idx_hbm, out_hbm, idx_v, rows_v, sem):
        wid = lax.axis_index("s") * N_CORES + lax.axis_index("c")
        base = wid * B_PER_W
        pltpu.sync_copy(idx_hbm.at[pl.ds(base, B_PER_W)], idx_v)

        @pl.loop(0, N_CHUNKS)
        def _(c):
            off = c * CHUNK
            pltpu.async_copy(
                table_hbm.at[idx_v.at[pl.ds(off, CHUNK)]], rows_v, sem
            ).wait()
            pltpu.sync_copy(rows_v, out_hbm.at[pl.ds(base + off, CHUNK)])

    return k(table, idx)


def kernel(positions, positional_embeddings):
    idx = positions.reshape(-1).astype(jnp.int32)
    out = _sc_gather(positional_embeddings, idx)
    return out.reshape(positions.shape + (positional_embeddings.shape[1],))


# speedup vs baseline: 2.3836x; 1.0884x over previous
import functools

import jax
import jax.numpy as jnp
from jax import lax
from jax.experimental import pallas as pl
from jax.experimental.pallas import tpu as pltpu
from jax.experimental.pallas import tpu_sc as plsc

N_CORES = 2
N_SUBCORES = 16
N_WORKERS = N_CORES * N_SUBCORES

D = 1024
B = 4 * 8192
B_PER_W = B // N_WORKERS
CHUNK = 16
NBUF = 4
N_CHUNKS = B_PER_W // CHUNK


def _sc_gather(table, idx):
    mesh = plsc.VectorSubcoreMesh(core_axis_name="c", subcore_axis_name="s")

    @functools.partial(
        pl.kernel,
        mesh=mesh,
        out_type=jax.ShapeDtypeStruct((B, D), jnp.float32),
        scratch_types=[
            pltpu.VMEM((B_PER_W,), jnp.int32),
            pltpu.VMEM((NBUF, CHUNK, D), jnp.float32),
        ]
        + [pltpu.SemaphoreType.DMA] * (2 * NBUF),
    )
    def k(table_hbm, idx_hbm, out_hbm, idx_v, rows_v, *sems):
        gsem, wsem = sems[:NBUF], sems[NBUF:]
        wid = lax.axis_index("s") * N_CORES + lax.axis_index("c")
        base = wid * B_PER_W
        pltpu.sync_copy(idx_hbm.at[pl.ds(base, B_PER_W)], idx_v)

        def gcopy(i, s):
            return pltpu.make_async_copy(
                table_hbm.at[idx_v.at[pl.ds(i * CHUNK, CHUNK)]],
                rows_v.at[s],
                gsem[s],
            )

        def wcopy(i, s):
            return pltpu.make_async_copy(
                rows_v.at[s],
                out_hbm.at[pl.ds(base + i * CHUNK, CHUNK)],
                wsem[s],
            )

        for m in range(NBUF - 1):
            gcopy(m, m).start()
        gcopy(0, 0).wait()
        wcopy(0, 0).start()
        gcopy(NBUF - 1, NBUF - 1).start()

        @pl.loop(0, (N_CHUNKS - NBUF) // NBUF)
        def _(blk):
            ibase = 1 + blk * NBUF
            for kk in range(NBUF):
                i = ibase + kk
                s = (1 + kk) % NBUF
                sp = (s - 1) % NBUF
                gcopy(i, s).wait()
                wcopy(i, s).start()
                wcopy(i - 1, sp).wait()
                gcopy(i + NBUF - 1, sp).start()

        for i in range(N_CHUNKS - NBUF + 1, N_CHUNKS):
            s = i % NBUF
            gcopy(i, s).wait()
            wcopy(i, s).start()
        for i in range(N_CHUNKS - NBUF, N_CHUNKS):
            wcopy(i, i % NBUF).wait()

    return k(table, idx)


def kernel(positions, positional_embeddings):
    idx = positions.reshape(-1).astype(jnp.int32)
    out = _sc_gather(positional_embeddings, idx)
    return out.reshape(positions.shape + (positional_embeddings.shape[1],))


# speedup vs baseline: 2.3987x; 1.0063x over previous
import functools

import jax
import jax.numpy as jnp
from jax import lax
from jax.experimental import pallas as pl
from jax.experimental.pallas import tpu as pltpu
from jax.experimental.pallas import tpu_sc as plsc

N_CORES = 2
N_SUBCORES = 16
N_WORKERS = N_CORES * N_SUBCORES

D = 1024
B = 4 * 8192
B_PER_W = B // N_WORKERS
CHUNK = 8
NBUF = 8
N_CHUNKS = B_PER_W // CHUNK


def _sc_gather(table, idx):
    mesh = plsc.VectorSubcoreMesh(core_axis_name="c", subcore_axis_name="s")

    @functools.partial(
        pl.kernel,
        mesh=mesh,
        out_type=jax.ShapeDtypeStruct((B, D), jnp.float32),
        scratch_types=[
            pltpu.VMEM((B_PER_W,), jnp.int32),
            pltpu.VMEM((NBUF, CHUNK, D), jnp.float32),
        ]
        + [pltpu.SemaphoreType.DMA] * (2 * NBUF),
    )
    def k(table_hbm, idx_hbm, out_hbm, idx_v, rows_v, *sems):
        gsem, wsem = sems[:NBUF], sems[NBUF:]
        wid = lax.axis_index("s") * N_CORES + lax.axis_index("c")
        base = wid * B_PER_W
        pltpu.sync_copy(idx_hbm.at[pl.ds(base, B_PER_W)], idx_v)

        def gcopy(i, s):
            return pltpu.make_async_copy(
                table_hbm.at[idx_v.at[pl.ds(i * CHUNK, CHUNK)]],
                rows_v.at[s],
                gsem[s],
            )

        def wcopy(i, s):
            return pltpu.make_async_copy(
                rows_v.at[s],
                out_hbm.at[pl.ds(base + i * CHUNK, CHUNK)],
                wsem[s],
            )

        for m in range(NBUF - 1):
            gcopy(m, m).start()
        gcopy(0, 0).wait()
        wcopy(0, 0).start()
        gcopy(NBUF - 1, NBUF - 1).start()

        @pl.loop(0, (N_CHUNKS - NBUF) // NBUF)
        def _(blk):
            ibase = 1 + blk * NBUF
            for kk in range(NBUF):
                i = ibase + kk
                s = (1 + kk) % NBUF
                sp = (s - 1) % NBUF
                gcopy(i, s).wait()
                wcopy(i, s).start()
                wcopy(i - 1, sp).wait()
                gcopy(i + NBUF - 1, sp).start()

        for i in range(N_CHUNKS - NBUF + 1, N_CHUNKS):
            s = i % NBUF
            gcopy(i, s).wait()
            wcopy(i, s).start()
        for i in range(N_CHUNKS - NBUF, N_CHUNKS):
            wcopy(i, i % NBUF).wait()

    return k(table, idx)


def kernel(positions, positional_embeddings):
    idx = positions.reshape(-1).astype(jnp.int32)
    out = _sc_gather(positional_embeddings, idx)
    return out.reshape(positions.shape + (positional_embeddings.shape[1],))
